# Initial kernel scaffold; baseline (speedup 1.0000x reference)
import jax
import jax.numpy as jnp
from jax.experimental import pallas as pl


def kernel(x, edge_index, W1, b1, W2, b2):
    raise NotImplementedError("write your pallas kernel here")


# speedup vs baseline: 12.9793x; 12.9793x over previous
import functools
import jax
import jax.numpy as jnp
from jax import lax
from jax.experimental import pallas as pl
from jax.experimental.pallas import tpu as pltpu
from jax.experimental.pallas import tpu_sc as plsc

N_CORES = 2
N_SUBCORES = 16
N_WORKERS = N_CORES * N_SUBCORES
CHUNK = 128

_MESH = functools.partial(
    plsc.VectorSubcoreMesh, core_axis_name="c", subcore_axis_name="s"
)


def _make_deg_kernel(n_pad, chunks):
  zrows = n_pad // N_SUBCORES

  @functools.partial(
      pl.kernel,
      mesh=_MESH(),
      out_type=jax.ShapeDtypeStruct((N_CORES, n_pad, 16), jnp.float32),
      scratch_types=[
          pltpu.VMEM((chunks, CHUNK), jnp.int32),
          pltpu.VMEM((CHUNK, 16), jnp.float32),
          pltpu.VMEM_SHARED((n_pad, 16), jnp.float32),
      ],
  )
  def deg_kernel(dst_hbm, ones_hbm, zeros_hbm, out_hbm, dst_v, ones_v, deg_sh):
    cid = lax.axis_index("c")
    sid = lax.axis_index("s")
    wid = cid * N_SUBCORES + sid
    zb = sid * zrows
    pltpu.sync_copy(zeros_hbm.at[pl.ds(zb, zrows)], deg_sh.at[pl.ds(zb, zrows)])
    pltpu.sync_copy(dst_hbm.at[wid], dst_v)
    pltpu.sync_copy(ones_hbm, ones_v)
    plsc.subcore_barrier()

    @pl.loop(0, chunks)
    def _(j):
      pltpu.sync_copy(ones_v, deg_sh.at[dst_v.at[j]], add=True)

    plsc.subcore_barrier()
    pltpu.sync_copy(
        deg_sh.at[pl.ds(zb, zrows)], out_hbm.at[cid, pl.ds(zb, zrows)]
    )

  return deg_kernel


def _make_agg_kernel(n_pad, chunks, feat):
  zrows = n_pad // N_SUBCORES

  @functools.partial(
      pl.kernel,
      mesh=_MESH(),
      out_type=jax.ShapeDtypeStruct((N_CORES, n_pad, feat), jnp.float32),
      scratch_types=[
          pltpu.VMEM((chunks, CHUNK), jnp.int32),
          pltpu.VMEM((chunks, CHUNK), jnp.int32),
          pltpu.VMEM((CHUNK, feat), jnp.float32),
          pltpu.VMEM_SHARED((n_pad, feat), jnp.float32),
      ],
  )
  def agg_kernel(
      hs_hbm, src_hbm, dst_hbm, zeros_hbm, out_hbm, src_v, dst_v, rows_v, acc_sh
  ):
    cid = lax.axis_index("c")
    sid = lax.axis_index("s")
    wid = cid * N_SUBCORES + sid
    zb = sid * zrows
    pltpu.sync_copy(zeros_hbm.at[pl.ds(zb, zrows)], acc_sh.at[pl.ds(zb, zrows)])
    pltpu.sync_copy(src_hbm.at[wid], src_v)
    pltpu.sync_copy(dst_hbm.at[wid], dst_v)
    plsc.subcore_barrier()

    @pl.loop(0, chunks)
    def _(j):
      pltpu.sync_copy(hs_hbm.at[src_v.at[j]], rows_v)
      pltpu.sync_copy(rows_v, acc_sh.at[dst_v.at[j]], add=True)

    plsc.subcore_barrier()
    pltpu.sync_copy(
        acc_sh.at[pl.ds(zb, zrows)], out_hbm.at[cid, pl.ds(zb, zrows)]
    )

  return agg_kernel


def _dis(deg_ref):
  deg = deg_ref[0, :, 0:1] + deg_ref[1, :, 0:1] + 1.0
  return lax.rsqrt(deg)


def _t1_body(deg_ref, x_ref, w_ref, hs_ref):
  dis = _dis(deg_ref)
  h = jnp.dot(x_ref[...], w_ref[...], preferred_element_type=jnp.float32)
  hs_ref[...] = h * dis


def _t2_body(deg_ref, agg_ref, hs1_ref, b1_ref, w2_ref, hs2_ref):
  dis = _dis(deg_ref)
  out1 = dis * (agg_ref[0] + agg_ref[1] + hs1_ref[...]) + b1_ref[...]
  h2 = jnp.maximum(out1, 0.0)
  hs2_ref[...] = (
      jnp.dot(h2, w2_ref[...], preferred_element_type=jnp.float32) * dis
  )


def _t3_body(deg_ref, agg_ref, hs_ref, b_ref, out_ref):
  dis = _dis(deg_ref)
  out_ref[...] = dis * (agg_ref[0] + agg_ref[1] + hs_ref[...]) + b_ref[...]


def kernel(x, edge_index, W1, b1, W2, b2):
  n, feat = x.shape
  e = edge_index.shape[1]
  src = edge_index[0].astype(jnp.int32)
  dst = edge_index[1].astype(jnp.int32)

  per_op = N_WORKERS * CHUNK
  chunks = -(-e // per_op)
  e_pad = chunks * per_op
  src_p = jnp.concatenate(
      [src, jnp.zeros((e_pad - e,), jnp.int32)]
  ).reshape(N_WORKERS, chunks, CHUNK)
  dst_p = jnp.concatenate(
      [dst, jnp.full((e_pad - e,), n, jnp.int32)]
  ).reshape(N_WORKERS, chunks, CHUNK)

  n_pad = -(-(n + 1) // 128) * 128
  zeros_f = jnp.zeros((n_pad, feat), jnp.float32)
  zeros_16 = jnp.zeros((n_pad, 16), jnp.float32)
  ones_16 = jnp.ones((CHUNK, 16), jnp.float32)

  deg2 = _make_deg_kernel(n_pad, chunks)(dst_p, ones_16, zeros_16)
  agg = _make_agg_kernel(n_pad, chunks, feat)

  blocks = 10
  bn = n // blocks
  deg_spec = pl.BlockSpec((N_CORES, bn, 16), lambda i: (0, i, 0))
  row_spec = pl.BlockSpec((bn, feat), lambda i: (i, 0))
  w_spec = pl.BlockSpec((feat, feat), lambda i: (0, 0))
  b_spec = pl.BlockSpec((1, feat), lambda i: (0, 0))
  agg_spec = pl.BlockSpec((N_CORES, bn, feat), lambda i: (0, i, 0))
  row_out = jax.ShapeDtypeStruct((n, feat), jnp.float32)

  hs1 = pl.pallas_call(
      _t1_body,
      grid=(blocks,),
      in_specs=[deg_spec, row_spec, w_spec],
      out_specs=row_spec,
      out_shape=row_out,
  )(deg2, x, W1)

  agg1 = agg(hs1, src_p, dst_p, zeros_f)

  hs2 = pl.pallas_call(
      _t2_body,
      grid=(blocks,),
      in_specs=[deg_spec, agg_spec, row_spec, b_spec, w_spec],
      out_specs=row_spec,
      out_shape=row_out,
  )(deg2, agg1, hs1, b1.reshape(1, feat), W2)

  agg2 = agg(hs2, src_p, dst_p, zeros_f)

  out = pl.pallas_call(
      _t3_body,
      grid=(blocks,),
      in_specs=[deg_spec, agg_spec, row_spec, b_spec],
      out_specs=row_spec,
      out_shape=row_out,
  )(deg2, agg2, hs2, b2.reshape(1, feat))

  return out
